# baseline (speedup 1.0000x reference)
import functools

import jax
import jax.numpy as jnp
from jax import lax
from jax.experimental import pallas as pl
from jax.experimental.pallas import tpu as pltpu
from jax.experimental.pallas import tpu_sc as plsc

T = 2048
D = 768
F = 3072
E = 8
TOPK = 2
TR = 256
NB = 24
RPAD = NB * TR
FT = 512
NF = F // FT
CH = 256
NCH = T // CH
NW = 32
CT = T // NW


def _route_body(x_ref, wg_ref, bg_ref, pos0_ref, pos1_ref, w0_ref, w1_ref,
                be_ref, bx_ref, nb_ref):
    f32 = jnp.float32
    x = x_ref[...]
    logits = jnp.dot(x, wg_ref[...], preferred_element_type=f32) + bg_ref[...]
    ie = lax.broadcasted_iota(jnp.int32, (T, E), 1)
    v1 = jnp.max(logits, axis=1, keepdims=True)
    e1 = jnp.min(jnp.where(logits >= v1, ie, E), axis=1, keepdims=True)
    m1 = ie == e1
    masked = jnp.where(m1, -jnp.inf, logits)
    v2 = jnp.max(masked, axis=1, keepdims=True)
    e2 = jnp.min(jnp.where(masked >= v2, ie, E), axis=1, keepdims=True)
    m2 = ie == e2
    r = jnp.exp(v2 - v1)
    w0_ref[...] = 1.0 / (1.0 + r)
    w1_ref[...] = r / (1.0 + r)
    A = m1.astype(f32) + m2.astype(f32)
    itc = lax.broadcasted_iota(jnp.int32, (NCH, T), 0)
    itt = lax.broadcasted_iota(jnp.int32, (NCH, T), 1)
    seg = (itc == itt // CH).astype(f32)
    csum = jnp.dot(seg, A, preferred_element_type=f32)
    r8 = lax.broadcasted_iota(jnp.int32, (NCH, NCH), 0)
    c8 = lax.broadcasted_iota(jnp.int32, (NCH, NCH), 1)
    slt8 = (c8 < r8).astype(f32)
    base = jnp.dot(slt8, csum, preferred_element_type=f32)
    rl = lax.broadcasted_iota(jnp.int32, (CH, CH), 0)
    cl = lax.broadcasted_iota(jnp.int32, (CH, CH), 1)
    sltc = (cl < rl).astype(f32)
    parts = []
    for c in range(NCH):
        Ac = A[c * CH:(c + 1) * CH, :]
        parts.append(jnp.dot(sltc, Ac, preferred_element_type=f32)
                     + base[c:c + 1, :])
    S = jnp.concatenate(parts, axis=0)
    g = jnp.sum(csum, axis=0, keepdims=True)
    blk = (g.astype(jnp.int32) + (TR - 1)) // TR
    blkf = blk.astype(f32)
    sut8 = (r8 < c8).astype(f32)
    bstartf = jnp.dot(blkf, sut8, preferred_element_type=f32)
    rowstart = bstartf * TR
    rank0 = jnp.sum(jnp.where(m1, S, 0.0), axis=1, keepdims=True)
    rank1 = jnp.sum(jnp.where(m2, S, 0.0), axis=1, keepdims=True)
    rs0 = jnp.sum(jnp.where(m1, rowstart, 0.0), axis=1, keepdims=True)
    rs1 = jnp.sum(jnp.where(m2, rowstart, 0.0), axis=1, keepdims=True)
    pos0_ref[...] = (rs0 + rank0).astype(jnp.int32)
    pos1_ref[...] = (rs1 + rank1).astype(jnp.int32)
    endf = bstartf + blkf
    bif = lax.broadcasted_iota(jnp.int32, (NB, E), 0).astype(f32)
    ownerf = jnp.sum((endf <= bif).astype(f32), axis=1, keepdims=True)
    totalf = jnp.sum(blkf)
    lastf = jnp.sum((endf <= totalf - 1.0).astype(f32))
    bi1 = lax.broadcasted_iota(jnp.int32, (NB, 1), 0).astype(f32)
    bef = jnp.where(bi1 < totalf, ownerf, lastf)
    be_ref[...] = jnp.clip(bef, 0.0, float(E - 1)).astype(jnp.int32)
    bx_ref[...] = jnp.minimum(bi1, totalf - 1.0).astype(jnp.int32)
    nb_ref[...] = jnp.full((1, 1), totalf, f32).astype(jnp.int32)


def _route(flat, Wg, bg):
    out = pl.pallas_call(
        _route_body,
        out_shape=(
            jax.ShapeDtypeStruct((T, 1), jnp.int32),
            jax.ShapeDtypeStruct((T, 1), jnp.int32),
            jax.ShapeDtypeStruct((T, 1), jnp.float32),
            jax.ShapeDtypeStruct((T, 1), jnp.float32),
            jax.ShapeDtypeStruct((NB, 1), jnp.int32),
            jax.ShapeDtypeStruct((NB, 1), jnp.int32),
            jax.ShapeDtypeStruct((1, 1), jnp.int32),
        ),
    )(flat, Wg, bg.reshape(1, E))
    return out


def _ffn_body(be_ref, bx_ref, nb_ref, x_ref, w1_ref, b1_ref, w2_ref, b2_ref,
              ws_ref, o_ref, acc_ref):
    b = pl.program_id(0)
    f = pl.program_id(1)

    @pl.when(b < nb_ref[0])
    def _():
        x = x_ref[...].astype(jnp.bfloat16)
        h = jnp.dot(x, w1_ref[0], preferred_element_type=jnp.float32)
        h = jax.nn.gelu(h + b1_ref[0])
        p = jnp.dot(h.astype(jnp.bfloat16), w2_ref[0],
                    preferred_element_type=jnp.float32)

        @pl.when(f == 0)
        def _():
            acc_ref[...] = p + b2_ref[0]

        @pl.when(f > 0)
        def _():
            acc_ref[...] += p

        @pl.when(f == NF - 1)
        def _():
            o_ref[...] = acc_ref[...] * ws_ref[...]


def _ffn(xs, ws, be, bx, nb, W1, b1, W2, b2):
    grid_spec = pltpu.PrefetchScalarGridSpec(
        num_scalar_prefetch=3,
        grid=(NB, NF),
        in_specs=[
            pl.BlockSpec((TR, D), lambda b, f, be, bx, nb: (bx[b], 0)),
            pl.BlockSpec((1, D, FT), lambda b, f, be, bx, nb: (be[b], 0, f)),
            pl.BlockSpec((1, 1, FT), lambda b, f, be, bx, nb: (be[b], 0, f)),
            pl.BlockSpec((1, FT, D), lambda b, f, be, bx, nb: (be[b], f, 0)),
            pl.BlockSpec((1, 1, D), lambda b, f, be, bx, nb: (be[b], 0, 0)),
            pl.BlockSpec((TR, 1), lambda b, f, be, bx, nb: (bx[b], 0)),
        ],
        out_specs=pl.BlockSpec((TR, D), lambda b, f, be, bx, nb: (bx[b], 0)),
        scratch_shapes=[pltpu.VMEM((TR, D), jnp.float32)],
    )
    return pl.pallas_call(
        _ffn_body,
        grid_spec=grid_spec,
        out_shape=jax.ShapeDtypeStruct((RPAD, D), jnp.float32),
        compiler_params=pltpu.CompilerParams(
            dimension_semantics=("arbitrary", "arbitrary")),
    )(be, bx, nb, xs, W1.astype(jnp.bfloat16), b1.reshape(E, 1, F),
      W2.astype(jnp.bfloat16), b2.reshape(E, 1, D), ws)


def _dispatch(flat, pos0, pos1, w0, w1):
    mesh = plsc.VectorSubcoreMesh(core_axis_name="c", subcore_axis_name="s")

    @functools.partial(
        pl.kernel,
        out_type=(
            jax.ShapeDtypeStruct((RPAD, D), jnp.float32),
            jax.ShapeDtypeStruct((RPAD,), jnp.float32),
        ),
        mesh=mesh,
        scratch_types=[
            pltpu.VMEM((CT, D), jnp.float32),
            pltpu.VMEM((CT,), jnp.int32),
            pltpu.VMEM((CT,), jnp.int32),
            pltpu.VMEM((T,), jnp.int32),
            pltpu.VMEM((T,), jnp.int32),
            pltpu.VMEM((T,), jnp.float32),
            pltpu.VMEM((T,), jnp.float32),
            pltpu.VMEM((RPAD,), jnp.float32),
            pltpu.SemaphoreType.DMA,
            pltpu.SemaphoreType.DMA,
        ],
        compiler_params=pltpu.CompilerParams(needs_layout_passes=False),
    )
    def k(flat_hbm, p0_hbm, p1_hbm, w0_hbm, w1_hbm, xs_hbm, ws_hbm,
          rows_v, i0_v, i1_v, p0_v, p1_v, w0_v, w1_v, ws_v, sem0, sem1):
        nc = plsc.get_sparse_core_info().num_cores
        wid = lax.axis_index("s") * nc + lax.axis_index("c")
        t0 = wid * CT
        pltpu.sync_copy(flat_hbm.at[pl.ds(t0, CT)], rows_v)
        pltpu.sync_copy(p0_hbm.at[pl.ds(t0, CT)], i0_v)
        pltpu.sync_copy(p1_hbm.at[pl.ds(t0, CT)], i1_v)
        c0 = pltpu.async_copy(rows_v, xs_hbm.at[i0_v], sem0)
        c1 = pltpu.async_copy(rows_v, xs_hbm.at[i1_v], sem1)

        @pl.when(wid == 0)
        def _():
            pltpu.sync_copy(p0_hbm, p0_v)
            pltpu.sync_copy(p1_hbm, p1_v)
            pltpu.sync_copy(w0_hbm, w0_v)
            pltpu.sync_copy(w1_hbm, w1_v)

            def body(i, carry):
                s = i * 16
                plsc.store_scatter(ws_v, [p0_v[pl.ds(s, 16)]],
                                   w0_v[pl.ds(s, 16)])
                plsc.store_scatter(ws_v, [p1_v[pl.ds(s, 16)]],
                                   w1_v[pl.ds(s, 16)])
                return carry

            lax.fori_loop(0, T // 16, body, 0)
            pltpu.sync_copy(ws_v, ws_hbm)

        c0.wait()
        c1.wait()

    return k(flat, pos0, pos1, w0, w1)


def _combine(ys, pos0, pos1):
    mesh = plsc.VectorSubcoreMesh(core_axis_name="c", subcore_axis_name="s")

    @functools.partial(
        pl.kernel,
        out_type=jax.ShapeDtypeStruct((T, D), jnp.float32),
        mesh=mesh,
        scratch_types=[
            pltpu.VMEM((CT, D), jnp.float32),
            pltpu.VMEM((CT, D), jnp.float32),
            pltpu.VMEM((CT,), jnp.int32),
            pltpu.VMEM((CT,), jnp.int32),
            pltpu.SemaphoreType.DMA,
            pltpu.SemaphoreType.DMA,
        ],
    )
    def k(ys_hbm, p0_hbm, p1_hbm, out_hbm, r0_v, r1_v, i0_v, i1_v, sem0, sem1):
        nc = plsc.get_sparse_core_info().num_cores
        wid = lax.axis_index("s") * nc + lax.axis_index("c")
        t0 = wid * CT
        pltpu.sync_copy(p0_hbm.at[pl.ds(t0, CT)], i0_v)
        pltpu.sync_copy(p1_hbm.at[pl.ds(t0, CT)], i1_v)
        c0 = pltpu.async_copy(ys_hbm.at[i0_v], r0_v, sem0)
        c1 = pltpu.async_copy(ys_hbm.at[i1_v], r1_v, sem1)
        c0.wait()
        c1.wait()

        def body(i, carry):
            for j in range(D // 16):
                sl = pl.ds(j * 16, 16)
                r0_v[i, sl] = r0_v[i, sl] + r1_v[i, sl]
            return carry

        lax.fori_loop(0, CT, body, 0)
        pltpu.sync_copy(r0_v, out_hbm.at[pl.ds(t0, CT)])

    return k(ys, pos0, pos1)


def kernel(x, Wg, bg, W1, b1, W2, b2):
    B, S, Dm = x.shape
    flat = x.reshape(B * S, Dm)
    pos0, pos1, w0, w1, be, bx, nb = _route(flat, Wg, bg)
    pos0 = pos0.reshape(T)
    pos1 = pos1.reshape(T)
    xs, ws = _dispatch(flat, pos0, pos1, w0.reshape(T), w1.reshape(T))
    ys = _ffn(xs, ws.reshape(RPAD, 1), be.reshape(NB), bx.reshape(NB),
              nb.reshape(1), W1, b1, W2, b2)
    out = _combine(ys, pos0, pos1)
    return out.reshape(B, S, Dm)


# speedup vs baseline: 1.7946x; 1.7946x over previous
import functools

import jax
import jax.numpy as jnp
from jax import lax
from jax.experimental import pallas as pl
from jax.experimental.pallas import tpu as pltpu
from jax.experimental.pallas import tpu_sc as plsc

T = 2048
D = 768
F = 3072
E = 8
TOPK = 2
TR = 256
NB = 24
RPAD = NB * TR
FT = 512
NF = F // FT
CH = 256
NCH = T // CH
NW = 32
CT = T // NW


def _route_body(x_ref, wg_ref, bg_ref, pos0_ref, pos1_ref, w0_ref, w1_ref,
                be_ref, bx_ref, nb_ref):
    f32 = jnp.float32
    x = x_ref[...]
    logits = jnp.dot(x, wg_ref[...], preferred_element_type=f32) + bg_ref[...]
    ie = lax.broadcasted_iota(jnp.int32, (T, E), 1)
    v1 = jnp.max(logits, axis=1, keepdims=True)
    e1 = jnp.min(jnp.where(logits >= v1, ie, E), axis=1, keepdims=True)
    m1 = ie == e1
    masked = jnp.where(m1, -jnp.inf, logits)
    v2 = jnp.max(masked, axis=1, keepdims=True)
    e2 = jnp.min(jnp.where(masked >= v2, ie, E), axis=1, keepdims=True)
    m2 = ie == e2
    r = jnp.exp(v2 - v1)
    w0_ref[...] = 1.0 / (1.0 + r)
    w1_ref[...] = r / (1.0 + r)
    A = m1.astype(f32) + m2.astype(f32)
    itc = lax.broadcasted_iota(jnp.int32, (NCH, T), 0)
    itt = lax.broadcasted_iota(jnp.int32, (NCH, T), 1)
    seg = (itc == itt // CH).astype(f32)
    csum = jnp.dot(seg, A, preferred_element_type=f32)
    r8 = lax.broadcasted_iota(jnp.int32, (NCH, NCH), 0)
    c8 = lax.broadcasted_iota(jnp.int32, (NCH, NCH), 1)
    slt8 = (c8 < r8).astype(f32)
    base = jnp.dot(slt8, csum, preferred_element_type=f32)
    rl = lax.broadcasted_iota(jnp.int32, (CH, CH), 0)
    cl = lax.broadcasted_iota(jnp.int32, (CH, CH), 1)
    sltc = (cl < rl).astype(f32)
    parts = []
    for c in range(NCH):
        Ac = A[c * CH:(c + 1) * CH, :]
        parts.append(jnp.dot(sltc, Ac, preferred_element_type=f32)
                     + base[c:c + 1, :])
    S = jnp.concatenate(parts, axis=0)
    g = jnp.sum(csum, axis=0, keepdims=True)
    blk = (g.astype(jnp.int32) + (TR - 1)) // TR
    blkf = blk.astype(f32)
    sut8 = (r8 < c8).astype(f32)
    bstartf = jnp.dot(blkf, sut8, preferred_element_type=f32)
    rowstart = bstartf * TR
    rank0 = jnp.sum(jnp.where(m1, S, 0.0), axis=1, keepdims=True)
    rank1 = jnp.sum(jnp.where(m2, S, 0.0), axis=1, keepdims=True)
    rs0 = jnp.sum(jnp.where(m1, rowstart, 0.0), axis=1, keepdims=True)
    rs1 = jnp.sum(jnp.where(m2, rowstart, 0.0), axis=1, keepdims=True)
    pos0_ref[...] = (rs0 + rank0).astype(jnp.int32)
    pos1_ref[...] = (rs1 + rank1).astype(jnp.int32)
    endf = bstartf + blkf
    bif = lax.broadcasted_iota(jnp.int32, (NB, E), 0).astype(f32)
    ownerf = jnp.sum((endf <= bif).astype(f32), axis=1, keepdims=True)
    totalf = jnp.sum(blkf)
    lastf = jnp.sum((endf <= totalf - 1.0).astype(f32))
    bi1 = lax.broadcasted_iota(jnp.int32, (NB, 1), 0).astype(f32)
    bef = jnp.where(bi1 < totalf, ownerf, lastf)
    be_ref[...] = jnp.clip(bef, 0.0, float(E - 1)).astype(jnp.int32)
    bx_ref[...] = jnp.minimum(bi1, totalf - 1.0).astype(jnp.int32)
    nb_ref[...] = jnp.full((1, 1), totalf, f32).astype(jnp.int32)


def _route(flat, Wg, bg):
    out = pl.pallas_call(
        _route_body,
        out_shape=(
            jax.ShapeDtypeStruct((T, 1), jnp.int32),
            jax.ShapeDtypeStruct((T, 1), jnp.int32),
            jax.ShapeDtypeStruct((T, 1), jnp.float32),
            jax.ShapeDtypeStruct((T, 1), jnp.float32),
            jax.ShapeDtypeStruct((NB, 1), jnp.int32),
            jax.ShapeDtypeStruct((NB, 1), jnp.int32),
            jax.ShapeDtypeStruct((1, 1), jnp.int32),
        ),
    )(flat, Wg, bg.reshape(1, E))
    return out


def _ffn_body(be_ref, bx_ref, nb_ref, x_ref, w1_ref, b1_ref, w2_ref, b2_ref,
              ws_ref, o_ref):
    b = pl.program_id(0)

    @pl.when(b < nb_ref[0])
    def _():
        x = x_ref[...].astype(jnp.bfloat16)
        w1 = w1_ref[0].astype(jnp.bfloat16)
        h = jnp.dot(x, w1, preferred_element_type=jnp.float32)
        h = jax.nn.gelu(h + b1_ref[0])
        w2 = w2_ref[0].astype(jnp.bfloat16)
        p = jnp.dot(h.astype(jnp.bfloat16), w2,
                    preferred_element_type=jnp.float32)
        o_ref[...] = (p + b2_ref[0]) * ws_ref[...]


def _ffn(xs, ws, be, bx, nb, W1, b1, W2, b2):
    grid_spec = pltpu.PrefetchScalarGridSpec(
        num_scalar_prefetch=3,
        grid=(NB,),
        in_specs=[
            pl.BlockSpec((TR, D), lambda b, be, bx, nb: (bx[b], 0)),
            pl.BlockSpec((1, D, F), lambda b, be, bx, nb: (be[b], 0, 0)),
            pl.BlockSpec((1, 1, F), lambda b, be, bx, nb: (be[b], 0, 0)),
            pl.BlockSpec((1, F, D), lambda b, be, bx, nb: (be[b], 0, 0)),
            pl.BlockSpec((1, 1, D), lambda b, be, bx, nb: (be[b], 0, 0)),
            pl.BlockSpec((TR, 1), lambda b, be, bx, nb: (bx[b], 0)),
        ],
        out_specs=pl.BlockSpec((TR, D), lambda b, be, bx, nb: (bx[b], 0)),
    )
    return pl.pallas_call(
        _ffn_body,
        grid_spec=grid_spec,
        out_shape=jax.ShapeDtypeStruct((RPAD, D), jnp.float32),
        compiler_params=pltpu.CompilerParams(
            dimension_semantics=("arbitrary",),
            vmem_limit_bytes=100 * 1024 * 1024),
    )(be, bx, nb, xs, W1, b1.reshape(E, 1, F), W2, b2.reshape(E, 1, D), ws)


def _dispatch(flat, pos0, pos1, w0, w1):
    mesh = plsc.VectorSubcoreMesh(core_axis_name="c", subcore_axis_name="s")

    @functools.partial(
        pl.kernel,
        out_type=(
            jax.ShapeDtypeStruct((RPAD, D), jnp.float32),
            jax.ShapeDtypeStruct((RPAD,), jnp.float32),
        ),
        mesh=mesh,
        scratch_types=[
            pltpu.VMEM((CT, D), jnp.float32),
            pltpu.VMEM((CT,), jnp.int32),
            pltpu.VMEM((CT,), jnp.int32),
            pltpu.VMEM((T,), jnp.int32),
            pltpu.VMEM((T,), jnp.int32),
            pltpu.VMEM((T,), jnp.float32),
            pltpu.VMEM((T,), jnp.float32),
            pltpu.VMEM((RPAD,), jnp.float32),
            pltpu.SemaphoreType.DMA,
            pltpu.SemaphoreType.DMA,
        ],
        compiler_params=pltpu.CompilerParams(needs_layout_passes=False),
    )
    def k(flat_hbm, p0_hbm, p1_hbm, w0_hbm, w1_hbm, xs_hbm, ws_hbm,
          rows_v, i0_v, i1_v, p0_v, p1_v, w0_v, w1_v, ws_v, sem0, sem1):
        nc = plsc.get_sparse_core_info().num_cores
        wid = lax.axis_index("s") * nc + lax.axis_index("c")
        t0 = wid * CT
        pltpu.sync_copy(flat_hbm.at[pl.ds(t0, CT)], rows_v)
        pltpu.sync_copy(p0_hbm.at[pl.ds(t0, CT)], i0_v)
        pltpu.sync_copy(p1_hbm.at[pl.ds(t0, CT)], i1_v)
        c0 = pltpu.async_copy(rows_v, xs_hbm.at[i0_v], sem0)
        c1 = pltpu.async_copy(rows_v, xs_hbm.at[i1_v], sem1)

        @pl.when(wid == 0)
        def _():
            pltpu.sync_copy(p0_hbm, p0_v)
            pltpu.sync_copy(p1_hbm, p1_v)
            pltpu.sync_copy(w0_hbm, w0_v)
            pltpu.sync_copy(w1_hbm, w1_v)

            def body(i, carry):
                s = i * 16
                plsc.store_scatter(ws_v, [p0_v[pl.ds(s, 16)]],
                                   w0_v[pl.ds(s, 16)])
                plsc.store_scatter(ws_v, [p1_v[pl.ds(s, 16)]],
                                   w1_v[pl.ds(s, 16)])
                return carry

            lax.fori_loop(0, T // 16, body, 0)
            pltpu.sync_copy(ws_v, ws_hbm)

        c0.wait()
        c1.wait()

    return k(flat, pos0, pos1, w0, w1)


def _combine(ys, pos0, pos1):
    mesh = plsc.VectorSubcoreMesh(core_axis_name="c", subcore_axis_name="s")

    @functools.partial(
        pl.kernel,
        out_type=jax.ShapeDtypeStruct((T, D), jnp.float32),
        mesh=mesh,
        scratch_types=[
            pltpu.VMEM((CT, D), jnp.float32),
            pltpu.VMEM((CT, D), jnp.float32),
            pltpu.VMEM((CT,), jnp.int32),
            pltpu.VMEM((CT,), jnp.int32),
            pltpu.SemaphoreType.DMA,
            pltpu.SemaphoreType.DMA,
        ],
    )
    def k(ys_hbm, p0_hbm, p1_hbm, out_hbm, r0_v, r1_v, i0_v, i1_v, sem0, sem1):
        nc = plsc.get_sparse_core_info().num_cores
        wid = lax.axis_index("s") * nc + lax.axis_index("c")
        t0 = wid * CT
        pltpu.sync_copy(p0_hbm.at[pl.ds(t0, CT)], i0_v)
        pltpu.sync_copy(p1_hbm.at[pl.ds(t0, CT)], i1_v)
        c0 = pltpu.async_copy(ys_hbm.at[i0_v], r0_v, sem0)
        c1 = pltpu.async_copy(ys_hbm.at[i1_v], r1_v, sem1)
        c0.wait()
        c1.wait()

        def body(i, carry):
            for j in range(D // 16):
                sl = pl.ds(j * 16, 16)
                r0_v[i, sl] = r0_v[i, sl] + r1_v[i, sl]
            return carry

        lax.fori_loop(0, CT, body, 0)
        pltpu.sync_copy(r0_v, out_hbm.at[pl.ds(t0, CT)])

    return k(ys, pos0, pos1)


def kernel(x, Wg, bg, W1, b1, W2, b2):
    B, S, Dm = x.shape
    flat = x.reshape(B * S, Dm)
    pos0, pos1, w0, w1, be, bx, nb = _route(flat, Wg, bg)
    pos0 = pos0.reshape(T)
    pos1 = pos1.reshape(T)
    xs, ws = _dispatch(flat, pos0, pos1, w0.reshape(T), w1.reshape(T))
    ys = _ffn(xs, ws.reshape(RPAD, 1), be.reshape(NB), bx.reshape(NB),
              nb.reshape(1), W1, b1, W2, b2)
    out = _combine(ys, pos0, pos1)
    return out.reshape(B, S, Dm)


# speedup vs baseline: 1.8683x; 1.0411x over previous
import functools

import jax
import jax.numpy as jnp
from jax import lax
from jax.experimental import pallas as pl
from jax.experimental.pallas import tpu as pltpu
from jax.experimental.pallas import tpu_sc as plsc

T = 2048
D = 768
F = 3072
E = 8
TOPK = 2
TR = 256
NB = 24
RPAD = NB * TR
FT = 512
NF = F // FT
CH = 256
NCH = T // CH
NW = 32
CT = T // NW


def _route_body(x_ref, wg_ref, bg_ref, pos0_ref, pos1_ref, w0_ref, w1_ref,
                be_ref, bx_ref, nb_ref):
    f32 = jnp.float32
    x = x_ref[...]
    logits = jnp.dot(x, wg_ref[...], preferred_element_type=f32) + bg_ref[...]
    ie = lax.broadcasted_iota(jnp.int32, (T, E), 1)
    v1 = jnp.max(logits, axis=1, keepdims=True)
    e1 = jnp.min(jnp.where(logits >= v1, ie, E), axis=1, keepdims=True)
    m1 = ie == e1
    masked = jnp.where(m1, -jnp.inf, logits)
    v2 = jnp.max(masked, axis=1, keepdims=True)
    e2 = jnp.min(jnp.where(masked >= v2, ie, E), axis=1, keepdims=True)
    m2 = ie == e2
    r = jnp.exp(v2 - v1)
    w0_ref[...] = (1.0 / (1.0 + r)).reshape(T)
    w1_ref[...] = (r / (1.0 + r)).reshape(T)
    A = m1.astype(f32) + m2.astype(f32)
    itc = lax.broadcasted_iota(jnp.int32, (NCH, T), 0)
    itt = lax.broadcasted_iota(jnp.int32, (NCH, T), 1)
    seg = (itc == itt // CH).astype(f32)
    csum = jnp.dot(seg, A, preferred_element_type=f32)
    r8 = lax.broadcasted_iota(jnp.int32, (NCH, NCH), 0)
    c8 = lax.broadcasted_iota(jnp.int32, (NCH, NCH), 1)
    slt8 = (c8 < r8).astype(f32)
    base = jnp.dot(slt8, csum, preferred_element_type=f32)
    rl = lax.broadcasted_iota(jnp.int32, (CH, CH), 0)
    cl = lax.broadcasted_iota(jnp.int32, (CH, CH), 1)
    sltc = (cl < rl).astype(f32)
    parts = []
    for c in range(NCH):
        Ac = A[c * CH:(c + 1) * CH, :]
        parts.append(jnp.dot(sltc, Ac, preferred_element_type=f32)
                     + base[c:c + 1, :])
    S = jnp.concatenate(parts, axis=0)
    g = jnp.sum(csum, axis=0, keepdims=True)
    blk = (g.astype(jnp.int32) + (TR - 1)) // TR
    blkf = blk.astype(f32)
    sut8 = (r8 < c8).astype(f32)
    bstartf = jnp.dot(blkf, sut8, preferred_element_type=f32)
    rowstart = bstartf * TR
    rank0 = jnp.sum(jnp.where(m1, S, 0.0), axis=1, keepdims=True)
    rank1 = jnp.sum(jnp.where(m2, S, 0.0), axis=1, keepdims=True)
    rs0 = jnp.sum(jnp.where(m1, rowstart, 0.0), axis=1, keepdims=True)
    rs1 = jnp.sum(jnp.where(m2, rowstart, 0.0), axis=1, keepdims=True)
    pos0_ref[...] = (rs0 + rank0).astype(jnp.int32).reshape(T)
    pos1_ref[...] = (rs1 + rank1).astype(jnp.int32).reshape(T)
    endf = bstartf + blkf
    bif = lax.broadcasted_iota(jnp.int32, (NB, E), 0).astype(f32)
    ownerf = jnp.sum((endf <= bif).astype(f32), axis=1, keepdims=True)
    totalf = jnp.sum(blkf)
    lastf = jnp.sum((endf <= totalf - 1.0).astype(f32))
    bi1 = lax.broadcasted_iota(jnp.int32, (NB, 1), 0).astype(f32)
    bef = jnp.where(bi1 < totalf, ownerf, lastf)
    be_ref[...] = jnp.clip(bef, 0.0, float(E - 1)).astype(jnp.int32)
    bx_ref[...] = jnp.minimum(bi1, totalf - 1.0).astype(jnp.int32)
    nb_ref[...] = jnp.full((1, 1), totalf, f32).astype(jnp.int32)


def _route(flat, Wg, bg):
    out = pl.pallas_call(
        _route_body,
        out_shape=(
            jax.ShapeDtypeStruct((T,), jnp.int32),
            jax.ShapeDtypeStruct((T,), jnp.int32),
            jax.ShapeDtypeStruct((T,), jnp.float32),
            jax.ShapeDtypeStruct((T,), jnp.float32),
            jax.ShapeDtypeStruct((NB, 1), jnp.int32),
            jax.ShapeDtypeStruct((NB, 1), jnp.int32),
            jax.ShapeDtypeStruct((1, 1), jnp.int32),
        ),
    )(flat, Wg, bg.reshape(1, E))
    return out


def _ffn_body(be_ref, bx_ref, nb_ref, x_ref, w1_ref, b1_ref, w2_ref, b2_ref,
              ws_ref, o_ref):
    b = pl.program_id(0)

    @pl.when(b < nb_ref[0])
    def _():
        x = x_ref[...].astype(jnp.bfloat16)
        w1 = w1_ref[0].astype(jnp.bfloat16)
        h = jnp.dot(x, w1, preferred_element_type=jnp.float32)
        h = jax.nn.gelu(h + b1_ref[0])
        w2 = w2_ref[0].astype(jnp.bfloat16)
        p = jnp.dot(h.astype(jnp.bfloat16), w2,
                    preferred_element_type=jnp.float32)
        o_ref[...] = (p + b2_ref[0]) * ws_ref[...]


def _ffn(xs, ws, be, bx, nb, W1, b1, W2, b2):
    grid_spec = pltpu.PrefetchScalarGridSpec(
        num_scalar_prefetch=3,
        grid=(NB,),
        in_specs=[
            pl.BlockSpec((TR, D), lambda b, be, bx, nb: (bx[b], 0)),
            pl.BlockSpec((1, D, F), lambda b, be, bx, nb: (be[b], 0, 0)),
            pl.BlockSpec((1, 1, F), lambda b, be, bx, nb: (be[b], 0, 0)),
            pl.BlockSpec((1, F, D), lambda b, be, bx, nb: (be[b], 0, 0)),
            pl.BlockSpec((1, 1, D), lambda b, be, bx, nb: (be[b], 0, 0)),
            pl.BlockSpec((TR, 1), lambda b, be, bx, nb: (bx[b], 0)),
        ],
        out_specs=pl.BlockSpec((TR, D), lambda b, be, bx, nb: (bx[b], 0)),
    )
    return pl.pallas_call(
        _ffn_body,
        grid_spec=grid_spec,
        out_shape=jax.ShapeDtypeStruct((RPAD, D), jnp.float32),
        compiler_params=pltpu.CompilerParams(
            dimension_semantics=("arbitrary",),
            vmem_limit_bytes=100 * 1024 * 1024),
    )(be, bx, nb, xs, W1, b1.reshape(E, 1, F), W2, b2.reshape(E, 1, D), ws)


def _dispatch(flat, pos0, pos1, w0, w1):
    mesh = plsc.VectorSubcoreMesh(core_axis_name="c", subcore_axis_name="s")

    @functools.partial(
        pl.kernel,
        out_type=(
            jax.ShapeDtypeStruct((RPAD, D), jnp.float32),
            jax.ShapeDtypeStruct((RPAD,), jnp.float32),
        ),
        mesh=mesh,
        scratch_types=[
            pltpu.VMEM((CT, D), jnp.float32),
            pltpu.VMEM((CT,), jnp.int32),
            pltpu.VMEM((CT,), jnp.int32),
            pltpu.VMEM((T,), jnp.int32),
            pltpu.VMEM((T,), jnp.int32),
            pltpu.VMEM((T,), jnp.float32),
            pltpu.VMEM((T,), jnp.float32),
            pltpu.VMEM((RPAD,), jnp.float32),
            pltpu.SemaphoreType.DMA,
            pltpu.SemaphoreType.DMA,
        ],
        compiler_params=pltpu.CompilerParams(needs_layout_passes=False),
    )
    def k(flat_hbm, p0_hbm, p1_hbm, w0_hbm, w1_hbm, xs_hbm, ws_hbm,
          rows_v, i0_v, i1_v, p0_v, p1_v, w0_v, w1_v, ws_v, sem0, sem1):
        nc = plsc.get_sparse_core_info().num_cores
        wid = lax.axis_index("s") * nc + lax.axis_index("c")
        t0 = wid * CT
        pltpu.sync_copy(flat_hbm.at[pl.ds(t0, CT)], rows_v)
        pltpu.sync_copy(p0_hbm.at[pl.ds(t0, CT)], i0_v)
        pltpu.sync_copy(p1_hbm.at[pl.ds(t0, CT)], i1_v)
        c0 = pltpu.async_copy(rows_v, xs_hbm.at[i0_v], sem0)
        c1 = pltpu.async_copy(rows_v, xs_hbm.at[i1_v], sem1)

        @pl.when(wid == 0)
        def _():
            pltpu.sync_copy(p0_hbm, p0_v)
            pltpu.sync_copy(p1_hbm, p1_v)
            pltpu.sync_copy(w0_hbm, w0_v)
            pltpu.sync_copy(w1_hbm, w1_v)

            def body(i, carry):
                s = i * 16
                plsc.store_scatter(ws_v, [p0_v[pl.ds(s, 16)]],
                                   w0_v[pl.ds(s, 16)])
                plsc.store_scatter(ws_v, [p1_v[pl.ds(s, 16)]],
                                   w1_v[pl.ds(s, 16)])
                return carry

            lax.fori_loop(0, T // 16, body, 0)
            pltpu.sync_copy(ws_v, ws_hbm)

        c0.wait()
        c1.wait()

    return k(flat, pos0, pos1, w0, w1)


def _combine(ys, pos0, pos1):
    mesh = plsc.VectorSubcoreMesh(core_axis_name="c", subcore_axis_name="s")

    @functools.partial(
        pl.kernel,
        out_type=jax.ShapeDtypeStruct((T, D), jnp.float32),
        mesh=mesh,
        scratch_types=[
            pltpu.VMEM((CT, D), jnp.float32),
            pltpu.VMEM((CT, D), jnp.float32),
            pltpu.VMEM((CT,), jnp.int32),
            pltpu.VMEM((CT,), jnp.int32),
            pltpu.SemaphoreType.DMA,
            pltpu.SemaphoreType.DMA,
        ],
    )
    def k(ys_hbm, p0_hbm, p1_hbm, out_hbm, r0_v, r1_v, i0_v, i1_v, sem0, sem1):
        nc = plsc.get_sparse_core_info().num_cores
        wid = lax.axis_index("s") * nc + lax.axis_index("c")
        t0 = wid * CT
        pltpu.sync_copy(p0_hbm.at[pl.ds(t0, CT)], i0_v)
        pltpu.sync_copy(p1_hbm.at[pl.ds(t0, CT)], i1_v)
        c0 = pltpu.async_copy(ys_hbm.at[i0_v], r0_v, sem0)
        c1 = pltpu.async_copy(ys_hbm.at[i1_v], r1_v, sem1)
        c0.wait()
        c1.wait()

        def body(i, carry):
            for j in range(D // 16):
                sl = pl.ds(j * 16, 16)
                r0_v[i, sl] = r0_v[i, sl] + r1_v[i, sl]
            return carry

        lax.fori_loop(0, CT, body, 0)
        pltpu.sync_copy(r0_v, out_hbm.at[pl.ds(t0, CT)])

    return k(ys, pos0, pos1)


def kernel(x, Wg, bg, W1, b1, W2, b2):
    B, S, Dm = x.shape
    flat = x.reshape(B * S, Dm)
    pos0, pos1, w0, w1, be, bx, nb = _route(flat, Wg, bg)
    xs, ws = _dispatch(flat, pos0, pos1, w0, w1)
    ys = _ffn(xs, ws.reshape(RPAD, 1), be.reshape(NB), bx.reshape(NB),
              nb.reshape(1), W1, b1, W2, b2)
    out = _combine(ys, pos0, pos1)
    return out.reshape(B, S, Dm)
